# baseline (speedup 1.0000x reference)
import functools

import jax
import jax.numpy as jnp
from jax import lax
from jax.experimental import pallas as pl
from jax.experimental.pallas import tpu as pltpu
from jax.experimental.pallas import tpu_sc as plsc

_E = 16
_K = 2
_NC = 2
_NS = 16
_NW = _NC * _NS
_BM = 1024


def _logits_body(x_ref, w_ref, o_ref):
    o_ref[...] = lax.dot_general(
        x_ref[...], w_ref[...],
        (((1,), (1,)), ((), ())),
        preferred_element_type=jnp.float32,
    )


def _logits(x, W):
    T, N = x.shape
    return pl.pallas_call(
        _logits_body,
        grid=(T // _BM,),
        in_specs=[
            pl.BlockSpec((_BM, N), lambda i: (i, 0)),
            pl.BlockSpec((_E, N), lambda i: (0, 0)),
        ],
        out_specs=pl.BlockSpec((_BM, _E), lambda i: (i, 0)),
        out_shape=jax.ShapeDtypeStruct((T, _E), jnp.float32),
    )(x, W)


def _route_body(rw, logits_hbm, w_hbm, i_hbm, lg_v, w_v, i_v):
    wid = lax.axis_index("s") * _NC + lax.axis_index("c")
    base = wid * rw
    pltpu.sync_copy(logits_hbm.at[pl.ds(base, rw)], lg_v)
    iota = lax.iota(jnp.int32, 16)
    zeros = jnp.zeros((16,), jnp.float32)

    @plsc.parallel_loop(0, rw // 16, unroll=2)
    def tile(t):
        row = t * 16 + iota
        ls = [
            plsc.load_gather(lg_v, [row, jnp.full((16,), e, jnp.int32)])
            for e in range(_E)
        ]
        m = ls[0]
        for e in range(1, _E):
            m = jnp.maximum(m, ls[e])
        es = [jnp.exp(l - m) for l in ls]
        s = es[0]
        for e in range(1, _E):
            s = s + es[e]
        inv = 1.0 / s
        m1 = es[0] * inv
        i1 = jnp.zeros((16,), jnp.int32)
        m2 = jnp.full((16,), -1.0, jnp.float32)
        i2 = jnp.zeros((16,), jnp.int32)
        for e in range(1, _E):
            p = es[e] * inv
            gt1 = p > m1
            gt2 = p > m2
            i2 = jnp.where(gt1, i1, jnp.where(gt2, e, i2))
            m2 = jnp.where(gt1, m1, jnp.where(gt2, p, m2))
            i1 = jnp.where(gt1, e, i1)
            m1 = jnp.where(gt1, p, m1)
        for j in range(16):
            w_v[t * 16 + j, :] = zeros
        plsc.store_scatter(w_v, [row, i1], m1)
        plsc.store_scatter(w_v, [row, i2], m2)
        plsc.store_scatter(i_v, [row * _K], i1)
        plsc.store_scatter(i_v, [row * _K + 1], i2)

    pltpu.sync_copy(w_v, w_hbm.at[pl.ds(base, rw)])
    pltpu.sync_copy(i_v, i_hbm.at[pl.ds(base * _K, rw * _K)])


def _route(logits):
    T = logits.shape[0]
    rw = T // _NW
    mesh = plsc.VectorSubcoreMesh(core_axis_name="c", subcore_axis_name="s")
    weights, idx_flat = pl.kernel(
        functools.partial(_route_body, rw),
        out_type=[
            jax.ShapeDtypeStruct((T, _E), jnp.float32),
            jax.ShapeDtypeStruct((T * _K,), jnp.int32),
        ],
        mesh=mesh,
        compiler_params=pltpu.CompilerParams(
            needs_layout_passes=False, skip_device_barrier=True
        ),
        scratch_types=[
            pltpu.VMEM((rw, _E), jnp.float32),
            pltpu.VMEM((rw, _E), jnp.float32),
            pltpu.VMEM((rw * _K,), jnp.int32),
        ],
    )(logits)
    return weights, idx_flat.reshape(T, _K)


def kernel(x, W):
    logits = _logits(x, W)
    return _route(logits)
